# baseline (speedup 1.0000x reference)
import jax
import jax.numpy as jnp
from jax.experimental import pallas as pl
from jax.experimental.pallas import tpu as pltpu

B, P, L, C = 16, 1024, 32, 9
H = 64
OUT = 64
N = B * P
L2 = L // 2
C2 = 2 * C
H2 = 2 * H

TILE = 256
RW = L2 * TILE


def _relu(x):
    return jnp.maximum(x, 0.0)


def _fused_body(x_ref, m_ref, pW1_ref, pb1_ref, pW2_ref, pb2_ref, pW3_ref,
                pb3_ref, mW1a_ref, mW1b_ref, mb1_ref, mW2_ref, mb2_ref,
                mW3_ref, mb3_ref, oW1_ref, ob1_ref, oW2_ref, ob2_ref,
                out_ref):
    f32 = jnp.float32
    bf = jnp.bfloat16
    x = x_ref[...].reshape(RW, C2).astype(bf)
    m = m_ref[...]

    mp = jnp.concatenate(
        [jnp.broadcast_to(m[:, :, 0:1], (TILE, L2, H)),
         jnp.broadcast_to(m[:, :, 1:2], (TILE, L2, H))],
        axis=-1).reshape(RW, H2)

    h = _relu(jnp.dot(x, pW1_ref[...], preferred_element_type=f32)
              + pb1_ref[...])
    h = _relu(jnp.dot(h.astype(bf), pW2_ref[...], preferred_element_type=f32)
              + pb2_ref[...])
    h = (jnp.dot(h.astype(bf), pW3_ref[...], preferred_element_type=f32)
         + pb3_ref[...])
    hm = h * mp

    pooled2 = jnp.max(hm.reshape(TILE, L2, H2), axis=1)
    pooled = jnp.maximum(pooled2[:, :H], pooled2[:, H:])

    pc = jnp.dot(pooled.astype(bf), mW1b_ref[...],
                 preferred_element_type=f32)
    pc2 = jnp.concatenate([pc, pc], axis=-1)
    pcb = jnp.broadcast_to(pc2[:, None, :], (TILE, L2, H2)).reshape(RW, H2)
    g = _relu(jnp.dot(hm.astype(bf), mW1a_ref[...],
                      preferred_element_type=f32) + pcb + mb1_ref[...])
    g = _relu(jnp.dot(g.astype(bf), mW2_ref[...], preferred_element_type=f32)
              + mb2_ref[...])
    g = (jnp.dot(g.astype(bf), mW3_ref[...], preferred_element_type=f32)
         + mb3_ref[...])
    gm = g * mp

    fb2 = jnp.max(gm.reshape(TILE, L2, H2), axis=1)
    fb = jnp.maximum(fb2[:, :H], fb2[:, H:])
    vm = jnp.max(m.reshape(TILE, L2 * 2), axis=1)[:, None]

    o = _relu(jnp.dot(fb.astype(bf), oW1_ref[...],
                      preferred_element_type=f32) + ob1_ref[...])
    o = (jnp.dot(o.astype(bf), oW2_ref[...], preferred_element_type=f32)
         + ob2_ref[...])
    out_ref[...] = o * vm


def _bdiag(W):
    k, n = W.shape
    z = jnp.zeros((k, n), W.dtype)
    return jnp.concatenate(
        [jnp.concatenate([W, z], axis=1),
         jnp.concatenate([z, W], axis=1)], axis=0)


def kernel(polylines, polylines_mask, pW1, pb1, pW2, pb2, pW3, pb3,
           mW1, mb1, mW2, mb2, mW3, mb3, oW1, ob1, oW2, ob2):
    bf = jnp.bfloat16
    x = polylines.reshape(N, L2, C2)
    m = polylines_mask.reshape(N, L2, 2).astype(jnp.float32)

    pW1d, pW2d, pW3d = _bdiag(pW1.astype(bf)), _bdiag(pW2.astype(bf)), \
        _bdiag(pW3.astype(bf))
    mW1a, mW1b = _bdiag(mW1[:H].astype(bf)), mW1[H:].astype(bf)
    mW2d, mW3d = _bdiag(mW2.astype(bf)), _bdiag(mW3.astype(bf))
    oW1b, oW2b = oW1.astype(bf), oW2.astype(bf)
    two = lambda b: jnp.concatenate([b, b]).reshape(1, H2)
    row = lambda b: b.reshape(1, -1)
    full = lambda s: pl.BlockSpec(s, lambda i: (0, 0))

    out = pl.pallas_call(
        _fused_body,
        grid=(N // TILE,),
        in_specs=[
            pl.BlockSpec((TILE, L2, C2), lambda i: (i, 0, 0)),
            pl.BlockSpec((TILE, L2, 2), lambda i: (i, 0, 0)),
            full((C2, H2)), full((1, H2)),
            full((H2, H2)), full((1, H2)),
            full((H2, H2)), full((1, H2)),
            full((H2, H2)), full((H, H)), full((1, H2)),
            full((H2, H2)), full((1, H2)),
            full((H2, H2)), full((1, H2)),
            full((H, H)), full((1, H)),
            full((H, OUT)), full((1, OUT)),
        ],
        out_specs=pl.BlockSpec((TILE, OUT), lambda i: (i, 0)),
        out_shape=jax.ShapeDtypeStruct((N, OUT), jnp.float32),
        compiler_params=pltpu.CompilerParams(
            dimension_semantics=("parallel",)),
    )(x, m, pW1d, two(pb1), pW2d, two(pb2), pW3d, two(pb3),
      mW1a, mW1b, two(mb1), mW2d, two(mb2), mW3d, two(mb3),
      oW1b, row(ob1), oW2b, row(ob2))
    return out.reshape(B, P, OUT)


# speedup vs baseline: 1.0916x; 1.0916x over previous
import jax
import jax.numpy as jnp
from jax.experimental import pallas as pl
from jax.experimental.pallas import tpu as pltpu

B, P, L, C = 16, 1024, 32, 9
H = 64
OUT = 64
N = B * P
L2 = L // 2
C2 = 2 * C
H2 = 2 * H
NQ = N * L2

TILE = 256
RW = L2 * TILE


def _relu(x):
    return jnp.maximum(x, jnp.zeros((), x.dtype))


def _fused_body(x_ref, m_ref, pW1_ref, pb1_ref, pW2_ref, pb2_ref, pW3_ref,
                pb3_ref, mW1a_ref, mW1b_ref, mb1_ref, mW2_ref, mb2_ref,
                mW3_ref, mb3_ref, oW1_ref, ob1_ref, oW2_ref, ob2_ref,
                out_ref):
    f32 = jnp.float32
    bf = jnp.bfloat16
    x = x_ref[...].T
    m3 = m_ref[...].T.reshape(TILE, L2, 2)

    mp = jnp.concatenate(
        [jnp.broadcast_to(m3[:, :, 0:1], (TILE, L2, H)),
         jnp.broadcast_to(m3[:, :, 1:2], (TILE, L2, H))],
        axis=-1).reshape(RW, H2)

    h = _relu(jnp.dot(x, pW1_ref[...],
                      preferred_element_type=f32).astype(bf) + pb1_ref[...])
    h = _relu(jnp.dot(h, pW2_ref[...],
                      preferred_element_type=f32).astype(bf) + pb2_ref[...])
    h = (jnp.dot(h, pW3_ref[...], preferred_element_type=f32).astype(bf)
         + pb3_ref[...])
    hm = h * mp

    pooled2 = jnp.max(hm.reshape(TILE, L2, H2), axis=1)
    pooled = jnp.maximum(pooled2[:, :H], pooled2[:, H:])

    pc = jnp.dot(pooled, mW1b_ref[...],
                 preferred_element_type=f32).astype(bf)
    pc2 = jnp.concatenate([pc, pc], axis=-1)
    pcb = jnp.broadcast_to(pc2[:, None, :], (TILE, L2, H2)).reshape(RW, H2)
    g = _relu(jnp.dot(hm, mW1a_ref[...],
                      preferred_element_type=f32).astype(bf)
              + pcb + mb1_ref[...])
    g = _relu(jnp.dot(g, mW2_ref[...],
                      preferred_element_type=f32).astype(bf) + mb2_ref[...])
    g = (jnp.dot(g, mW3_ref[...], preferred_element_type=f32).astype(bf)
         + mb3_ref[...])
    gm = g * mp

    fb2 = jnp.max(gm.reshape(TILE, L2, H2), axis=1)
    fb = jnp.maximum(fb2[:, :H], fb2[:, H:])
    vm = jnp.max(jnp.max(m3, axis=1), axis=1, keepdims=True)

    o = _relu(jnp.dot(fb, oW1_ref[...], preferred_element_type=f32)
              + ob1_ref[...])
    o = (jnp.dot(o.astype(bf), oW2_ref[...], preferred_element_type=f32)
         + ob2_ref[...])
    out_ref[...] = o * vm.astype(f32)


def _bdiag(W):
    k, n = W.shape
    z = jnp.zeros((k, n), W.dtype)
    return jnp.concatenate(
        [jnp.concatenate([W, z], axis=1),
         jnp.concatenate([z, W], axis=1)], axis=0)


def kernel(polylines, polylines_mask, pW1, pb1, pW2, pb2, pW3, pb3,
           mW1, mb1, mW2, mb2, mW3, mb3, oW1, ob1, oW2, ob2):
    bf = jnp.bfloat16
    x = (polylines.astype(bf).reshape(N, L2, 2, C)
         .transpose(2, 3, 0, 1).reshape(C2, NQ))
    m = (polylines_mask.reshape(N, L2, 2).transpose(2, 0, 1)
         .reshape(2, NQ).astype(bf))

    pW1d, pW2d, pW3d = _bdiag(pW1.astype(bf)), _bdiag(pW2.astype(bf)), \
        _bdiag(pW3.astype(bf))
    mW1a, mW1b = _bdiag(mW1[:H].astype(bf)), mW1[H:].astype(bf)
    mW2d, mW3d = _bdiag(mW2.astype(bf)), _bdiag(mW3.astype(bf))
    oW1b, oW2b = oW1.astype(bf), oW2.astype(bf)
    two = lambda b: jnp.concatenate([b, b]).reshape(1, H2).astype(bf)
    row = lambda b: b.reshape(1, -1)
    full = lambda s: pl.BlockSpec(s, lambda i: (0, 0))

    out = pl.pallas_call(
        _fused_body,
        grid=(N // TILE,),
        in_specs=[
            pl.BlockSpec((C2, RW), lambda i: (0, i)),
            pl.BlockSpec((2, RW), lambda i: (0, i)),
            full((C2, H2)), full((1, H2)),
            full((H2, H2)), full((1, H2)),
            full((H2, H2)), full((1, H2)),
            full((H2, H2)), full((H, H)), full((1, H2)),
            full((H2, H2)), full((1, H2)),
            full((H2, H2)), full((1, H2)),
            full((H, H)), full((1, H)),
            full((H, OUT)), full((1, OUT)),
        ],
        out_specs=pl.BlockSpec((TILE, OUT), lambda i: (i, 0)),
        out_shape=jax.ShapeDtypeStruct((N, OUT), jnp.float32),
        compiler_params=pltpu.CompilerParams(
            dimension_semantics=("parallel",)),
    )(x, m, pW1d, two(pb1), pW2d, two(pb2), pW3d, two(pb3),
      mW1a, mW1b, two(mb1), mW2d, two(mb2), mW3d, two(mb3),
      oW1b, row(ob1), oW2b, row(ob2))
    return out.reshape(B, P, OUT)
